# Initial kernel scaffold; baseline (speedup 1.0000x reference)
import jax
import jax.numpy as jnp
from jax.experimental import pallas as pl


def kernel(s_idx, r_idx, o_idx, t_idx, ent_re, ent_im, rel_re, rel_im, time_re, time_im):
    raise NotImplementedError("write your pallas kernel here")


# speedup vs baseline: 3.0588x; 3.0588x over previous
import jax
import jax.numpy as jnp
from jax import lax
from jax.experimental import pallas as pl
from jax.experimental.pallas import tpu as pltpu
from jax.experimental.pallas import tpu_sc as plsc

_B = 16384
_D = 128
_NC = 2
_NS = 16
_NW = _NC * _NS
_CPW = _B // _NW
_G = 64
_NCH = _CPW // _G

_GATHER_DNUMS = lax.GatherDimensionNumbers(
    offset_dims=(), collapsed_slice_dims=(0,), start_index_map=(0,))


def _xlane(v, idx):
  return lax.gather(v, idx[:, None], _GATHER_DNUMS, slice_sizes=(1,),
                    mode=lax.GatherScatterMode.PROMISE_IN_BOUNDS)


def _tcomplex_body(s_idx, r_idx, o_idx, t_idx,
                   ent_re, ent_im, rel_re, rel_im, time_re, time_im,
                   out,
                   si_v, ri_v, oi_v, ti_v,
                   sre_v, sim_v, rre_v, rim_v, ore_v, oim_v, tre_v, tim_v,
                   out_v, sem):
  wid = lax.axis_index("s") * _NC + lax.axis_index("c")
  base = wid * _CPW
  lane = lax.iota(jnp.int32, 16)
  rots = [(lane + sh) % 16 for sh in (8, 4, 2, 1)]
  lane_masks = [lane == r for r in range(16)]

  for k in range(_NCH):
    off = pl.multiple_of(base + k * _G, _G)
    pltpu.sync_copy(s_idx.at[pl.ds(off, _G)], si_v)
    pltpu.sync_copy(r_idx.at[pl.ds(off, _G)], ri_v)
    pltpu.sync_copy(o_idx.at[pl.ds(off, _G)], oi_v)
    pltpu.sync_copy(t_idx.at[pl.ds(off, _G)], ti_v)
    cps = [
        pltpu.async_copy(ent_re.at[si_v], sre_v, sem),
        pltpu.async_copy(ent_im.at[si_v], sim_v, sem),
        pltpu.async_copy(rel_re.at[ri_v], rre_v, sem),
        pltpu.async_copy(rel_im.at[ri_v], rim_v, sem),
        pltpu.async_copy(ent_re.at[oi_v], ore_v, sem),
        pltpu.async_copy(ent_im.at[oi_v], oim_v, sem),
        pltpu.async_copy(time_re.at[ti_v], tre_v, sem),
        pltpu.async_copy(time_im.at[ti_v], tim_v, sem),
    ]
    for cp in cps:
      cp.wait()

    def group16(j, carry):
      jbase = j * 16

      def row(r, tot):
        i = jbase + r
        acc = jnp.zeros((16,), jnp.float32)
        for d in range(8):
          sl = pl.ds(d * 16, 16)
          sre = sre_v[i, sl]
          sim = sim_v[i, sl]
          ore = ore_v[i, sl]
          oim = oim_v[i, sl]
          rre = rre_v[i, sl]
          rim = rim_v[i, sl]
          tre = tre_v[i, sl]
          tim = tim_v[i, sl]
          rt_re = rre * tre - rim * tim
          rt_im = rre * tim + rim * tre
          acc = acc + sre * (rt_re * ore + rt_im * oim) \
                    + sim * (rt_re * oim - rt_im * ore)
        for rv in rots:
          acc = acc + _xlane(acc, rv)
        return jnp.where(lane == r, acc, tot)

      tot = lax.fori_loop(0, 16, row, jnp.zeros((16,), jnp.float32))
      out_v[pl.ds(jbase, 16)] = tot
      return carry

    lax.fori_loop(0, _G // 16, group16, 0)
    pltpu.sync_copy(out_v, out.at[pl.ds(off, _G)])


def kernel(s_idx, r_idx, o_idx, t_idx,
           ent_re, ent_im, rel_re, rel_im, time_re, time_im):
  mesh = plsc.VectorSubcoreMesh(core_axis_name="c", subcore_axis_name="s")
  f = pl.kernel(
      _tcomplex_body,
      out_type=jax.ShapeDtypeStruct((_B,), jnp.float32),
      mesh=mesh,
      scratch_types=[
          pltpu.VMEM((_G,), jnp.int32),
          pltpu.VMEM((_G,), jnp.int32),
          pltpu.VMEM((_G,), jnp.int32),
          pltpu.VMEM((_G,), jnp.int32),
          pltpu.VMEM((_G, _D), jnp.float32),
          pltpu.VMEM((_G, _D), jnp.float32),
          pltpu.VMEM((_G, _D), jnp.float32),
          pltpu.VMEM((_G, _D), jnp.float32),
          pltpu.VMEM((_G, _D), jnp.float32),
          pltpu.VMEM((_G, _D), jnp.float32),
          pltpu.VMEM((_G, _D), jnp.float32),
          pltpu.VMEM((_G, _D), jnp.float32),
          pltpu.VMEM((_G,), jnp.float32),
          pltpu.SemaphoreType.DMA,
      ],
  )
  return f(s_idx.astype(jnp.int32), r_idx.astype(jnp.int32),
           o_idx.astype(jnp.int32), t_idx.astype(jnp.int32),
           ent_re, ent_im, rel_re, rel_im, time_re, time_im)


# speedup vs baseline: 3.3576x; 1.0977x over previous
import jax
import jax.numpy as jnp
from jax import lax
from jax.experimental import pallas as pl
from jax.experimental.pallas import tpu as pltpu
from jax.experimental.pallas import tpu_sc as plsc

_B = 16384
_D = 128
_NC = 2
_NS = 16
_NW = _NC * _NS
_CPW = _B // _NW
_G = 32
_NCH = _CPW // _G

_GATHER_DNUMS = lax.GatherDimensionNumbers(
    offset_dims=(), collapsed_slice_dims=(0,), start_index_map=(0,))


def _xlane(v, idx):
  return lax.gather(v, idx[:, None], _GATHER_DNUMS, slice_sizes=(1,),
                    mode=lax.GatherScatterMode.PROMISE_IN_BOUNDS)


def _tcomplex_body(s_idx, r_idx, o_idx, t_idx,
                   ent_re, ent_im, rel_re, rel_im, time_re, time_im,
                   out,
                   si0, ri0, oi0, ti0, si1, ri1, oi1, ti1,
                   sre0, sim0, rre0, rim0, ore0, oim0, tre0, tim0,
                   sre1, sim1, rre1, rim1, ore1, oim1, tre1, tim1,
                   out_v, sem0, sem1):
  wid = lax.axis_index("s") * _NC + lax.axis_index("c")
  base = wid * _CPW
  lane = lax.iota(jnp.int32, 16)
  rots = [(lane + sh) % 16 for sh in (8, 4, 2, 1)]

  idx_bufs = [(si0, ri0, oi0, ti0), (si1, ri1, oi1, ti1)]
  row_bufs = [(sre0, sim0, rre0, rim0, ore0, oim0, tre0, tim0),
              (sre1, sim1, rre1, rim1, ore1, oim1, tre1, tim1)]
  sems = [sem0, sem1]

  def prefetch(k):
    b = k % 2
    off = pl.multiple_of(base + k * _G, _G)
    si, ri, oi, ti = idx_bufs[b]
    pltpu.sync_copy(s_idx.at[pl.ds(off, _G)], si)
    pltpu.sync_copy(r_idx.at[pl.ds(off, _G)], ri)
    pltpu.sync_copy(o_idx.at[pl.ds(off, _G)], oi)
    pltpu.sync_copy(t_idx.at[pl.ds(off, _G)], ti)
    sre, sim, rre, rim, ore, oim, tre, tim = row_bufs[b]
    sem = sems[b]
    return [
        pltpu.async_copy(ent_re.at[si], sre, sem),
        pltpu.async_copy(ent_im.at[si], sim, sem),
        pltpu.async_copy(rel_re.at[ri], rre, sem),
        pltpu.async_copy(rel_im.at[ri], rim, sem),
        pltpu.async_copy(ent_re.at[oi], ore, sem),
        pltpu.async_copy(ent_im.at[oi], oim, sem),
        pltpu.async_copy(time_re.at[ti], tre, sem),
        pltpu.async_copy(time_im.at[ti], tim, sem),
    ]

  def compute(k):
    b = k % 2
    sre_v, sim_v, rre_v, rim_v, ore_v, oim_v, tre_v, tim_v = row_bufs[b]

    def group16(j, carry):
      jbase = j * 16

      def row(r, tot):
        i = jbase + r
        acc = jnp.zeros((16,), jnp.float32)
        for d in range(8):
          sl = pl.ds(d * 16, 16)
          sre = sre_v[i, sl]
          sim = sim_v[i, sl]
          ore = ore_v[i, sl]
          oim = oim_v[i, sl]
          rre = rre_v[i, sl]
          rim = rim_v[i, sl]
          tre = tre_v[i, sl]
          tim = tim_v[i, sl]
          rt_re = rre * tre - rim * tim
          rt_im = rre * tim + rim * tre
          acc = acc + sre * (rt_re * ore + rt_im * oim) \
                    + sim * (rt_re * oim - rt_im * ore)
        for rv in rots:
          acc = acc + _xlane(acc, rv)
        return jnp.where(lane == r, acc, tot)

      tot = lax.fori_loop(0, 16, row, jnp.zeros((16,), jnp.float32))
      out_v[pl.ds(jbase, 16)] = tot
      return carry

    lax.fori_loop(0, _G // 16, group16, 0)
    off = pl.multiple_of(base + k * _G, _G)
    pltpu.sync_copy(out_v, out.at[pl.ds(off, _G)])

  cps = prefetch(0)
  for k in range(_NCH):
    nxt = prefetch(k + 1) if k + 1 < _NCH else []
    for cp in cps:
      cp.wait()
    compute(k)
    cps = nxt


def kernel(s_idx, r_idx, o_idx, t_idx,
           ent_re, ent_im, rel_re, rel_im, time_re, time_im):
  mesh = plsc.VectorSubcoreMesh(core_axis_name="c", subcore_axis_name="s")
  idx_t = pltpu.VMEM((_G,), jnp.int32)
  row_t = pltpu.VMEM((_G, _D), jnp.float32)
  f = pl.kernel(
      _tcomplex_body,
      out_type=jax.ShapeDtypeStruct((_B,), jnp.float32),
      mesh=mesh,
      scratch_types=(
          [idx_t] * 8 + [row_t] * 16
          + [pltpu.VMEM((_G,), jnp.float32),
             pltpu.SemaphoreType.DMA, pltpu.SemaphoreType.DMA]
      ),
  )
  return f(s_idx.astype(jnp.int32), r_idx.astype(jnp.int32),
           o_idx.astype(jnp.int32), t_idx.astype(jnp.int32),
           ent_re, ent_im, rel_re, rel_im, time_re, time_im)


# speedup vs baseline: 4.2462x; 1.2647x over previous
import jax
import jax.numpy as jnp
from jax import lax
from jax.experimental import pallas as pl
from jax.experimental.pallas import tpu as pltpu
from jax.experimental.pallas import tpu_sc as plsc

_B = 16384
_D = 128
_NC = 2
_NS = 16
_NW = _NC * _NS
_CPW = _B // _NW
_G = 32
_NCH = _CPW // _G

_GATHER_DNUMS = lax.GatherDimensionNumbers(
    offset_dims=(), collapsed_slice_dims=(0,), start_index_map=(0,))


def _xlane(v, idx):
  return lax.gather(v, idx[:, None], _GATHER_DNUMS, slice_sizes=(1,),
                    mode=lax.GatherScatterMode.PROMISE_IN_BOUNDS)


def _tcomplex_body(s_idx, r_idx, o_idx, t_idx,
                   ent_re, ent_im, rel_re, rel_im, time_re, time_im,
                   out,
                   si_all, ri_all, oi_all, ti_all,
                   sre0, sim0, rre0, rim0, ore0, oim0, tre0, tim0,
                   sre1, sim1, rre1, rim1, ore1, oim1, tre1, tim1,
                   out_v, sem0, sem1):
  wid = lax.axis_index("s") * _NC + lax.axis_index("c")
  base = pl.multiple_of(wid * _CPW, _CPW)
  lane = lax.iota(jnp.int32, 16)
  rots = [(lane + sh) % 16 for sh in (8, 4, 2, 1)]

  row_bufs = [(sre0, sim0, rre0, rim0, ore0, oim0, tre0, tim0),
              (sre1, sim1, rre1, rim1, ore1, oim1, tre1, tim1)]
  sems = [sem0, sem1]

  pltpu.sync_copy(s_idx.at[pl.ds(base, _CPW)], si_all)
  pltpu.sync_copy(r_idx.at[pl.ds(base, _CPW)], ri_all)
  pltpu.sync_copy(o_idx.at[pl.ds(base, _CPW)], oi_all)
  pltpu.sync_copy(t_idx.at[pl.ds(base, _CPW)], ti_all)

  def prefetch(k):
    b = k % 2
    sl = pl.ds(k * _G, _G)
    sre, sim, rre, rim, ore, oim, tre, tim = row_bufs[b]
    sem = sems[b]
    return [
        pltpu.async_copy(ent_re.at[si_all.at[sl]], sre, sem),
        pltpu.async_copy(ent_im.at[si_all.at[sl]], sim, sem),
        pltpu.async_copy(rel_re.at[ri_all.at[sl]], rre, sem),
        pltpu.async_copy(rel_im.at[ri_all.at[sl]], rim, sem),
        pltpu.async_copy(ent_re.at[oi_all.at[sl]], ore, sem),
        pltpu.async_copy(ent_im.at[oi_all.at[sl]], oim, sem),
        pltpu.async_copy(time_re.at[ti_all.at[sl]], tre, sem),
        pltpu.async_copy(time_im.at[ti_all.at[sl]], tim, sem),
    ]

  def compute(k):
    b = k % 2
    sre_v, sim_v, rre_v, rim_v, ore_v, oim_v, tre_v, tim_v = row_bufs[b]

    def group16(j, carry):
      jbase = j * 16

      def row(r, tot):
        i = jbase + r
        acc = jnp.zeros((16,), jnp.float32)
        for d in range(8):
          sl = pl.ds(d * 16, 16)
          sre = sre_v[i, sl]
          sim = sim_v[i, sl]
          ore = ore_v[i, sl]
          oim = oim_v[i, sl]
          rre = rre_v[i, sl]
          rim = rim_v[i, sl]
          tre = tre_v[i, sl]
          tim = tim_v[i, sl]
          rt_re = rre * tre - rim * tim
          rt_im = rre * tim + rim * tre
          acc = acc + sre * (rt_re * ore + rt_im * oim) \
                    + sim * (rt_re * oim - rt_im * ore)
        for rv in rots:
          acc = acc + _xlane(acc, rv)
        return jnp.where(lane == r, acc, tot)

      tot = lax.fori_loop(0, 16, row, jnp.zeros((16,), jnp.float32))
      out_v[pl.ds(k * _G + jbase, 16)] = tot
      return carry

    lax.fori_loop(0, _G // 16, group16, 0)

  cps = prefetch(0)
  for k in range(_NCH):
    nxt = prefetch(k + 1) if k + 1 < _NCH else []
    for cp in cps:
      cp.wait()
    compute(k)
    cps = nxt
  pltpu.sync_copy(out_v, out.at[pl.ds(base, _CPW)])


def kernel(s_idx, r_idx, o_idx, t_idx,
           ent_re, ent_im, rel_re, rel_im, time_re, time_im):
  mesh = plsc.VectorSubcoreMesh(core_axis_name="c", subcore_axis_name="s")
  idx_t = pltpu.VMEM((_CPW,), jnp.int32)
  row_t = pltpu.VMEM((_G, _D), jnp.float32)
  f = pl.kernel(
      _tcomplex_body,
      out_type=jax.ShapeDtypeStruct((_B,), jnp.float32),
      mesh=mesh,
      scratch_types=(
          [idx_t] * 4 + [row_t] * 16
          + [pltpu.VMEM((_CPW,), jnp.float32),
             pltpu.SemaphoreType.DMA, pltpu.SemaphoreType.DMA]
      ),
  )
  return f(s_idx.astype(jnp.int32), r_idx.astype(jnp.int32),
           o_idx.astype(jnp.int32), t_idx.astype(jnp.int32),
           ent_re, ent_im, rel_re, rel_im, time_re, time_im)
